# baseline (speedup 1.0000x reference)
import functools
import math

import jax
import jax.numpy as jnp
from jax import lax
from jax.experimental import pallas as pl
from jax.experimental.pallas import tpu as pltpu
from jax.experimental.pallas import tpu_sc as plsc


def _uv_body(W, H, ld_ref, i00_ref, i01_ref, i10_ref, i11_ref,
             e0_ref, e1_ref, wx_ref, wy_ref):
    x = ld_ref[0, :]
    y = ld_ref[1, :]
    z = ld_ref[2, :]
    norm = jnp.sqrt(x * x + y * y + z * z)
    yn = y / norm
    yc = jnp.clip(yn, -1.0 + 1e-06, 1.0 - 1e-06)
    theta = jnp.arctan2(jnp.sqrt((1.0 + yc) * (1.0 - yc)), yc)
    phi = jnp.arctan2(x, z)
    u = phi / (2.0 * math.pi) + 0.5
    u = u - jnp.floor(u)
    v = theta / math.pi
    xf = jnp.clip(u * W, 0.0, W - 1.0)
    yf = jnp.clip(v * H, 0.0, H - 1.0)
    x0f = jnp.floor(xf)
    y0f = jnp.floor(yf)
    wx_ref[...] = xf - x0f
    wy_ref[...] = yf - y0f
    x0 = x0f.astype(jnp.int32)
    y0 = y0f.astype(jnp.int32)
    x1 = jnp.minimum(x0 + 1, int(W) - 1)
    y1 = jnp.minimum(y0 + 1, int(H) - 1)
    W4 = int(W) // 4
    k = x0 >> 2
    kb = jnp.minimum(k + 1, W4 - 1)
    i00_ref[...] = y0 * W4 + k
    i01_ref[...] = y0 * W4 + kb
    i10_ref[...] = y1 * W4 + k
    i11_ref[...] = y1 * W4 + kb
    e0_ref[...] = x0 & 3
    e1_ref[...] = x1 - (k << 2)


def _uv_kernel(ldT, H, W, TB=8192):
    B = ldT.shape[1]
    G = B // TB
    iout = jax.ShapeDtypeStruct((B,), jnp.int32)
    fout = jax.ShapeDtypeStruct((B,), jnp.float32)
    ospec = pl.BlockSpec((TB,), lambda i: (i,))
    outs = pl.pallas_call(
        functools.partial(_uv_body, float(W), float(H)),
        grid=(G,),
        in_specs=[pl.BlockSpec((3, TB), lambda i: (0, i))],
        out_specs=[ospec] * 8,
        out_shape=[iout, iout, iout, iout, iout, iout, fout, fout],
    )(ldT)
    return outs


_LANES = 16


def _sc_gather_combine(env16, i00, i01, i10, i11, e0, e1, wx, wy, C=512):
    B = i00.shape[0]
    info = plsc.get_sparse_core_info()
    NC, NS = info.num_cores, info.num_subcores
    NW = NC * NS
    RW = B // NW
    NCHUNK = RW // C
    GROUPS = C // _LANES
    assert NCHUNK % 2 == 0

    mesh = plsc.VectorSubcoreMesh(core_axis_name="c", subcore_axis_name="s")
    fout = jax.ShapeDtypeStruct((B,), jnp.float32)

    buf_set = [
        pltpu.VMEM((4, C), jnp.int32),
        pltpu.VMEM((2, C), jnp.int32),
        pltpu.VMEM((4 * C, 16), jnp.float32),
        pltpu.VMEM((2, C), jnp.float32),
    ]

    @functools.partial(
        pl.kernel,
        out_type=[fout, fout, fout],
        mesh=mesh,
        scratch_types=buf_set + buf_set + [
            pltpu.VMEM((3, C), jnp.float32),
            pltpu.SemaphoreType.DMA,
            pltpu.SemaphoreType.DMA,
            pltpu.SemaphoreType.DMA,
            pltpu.SemaphoreType.DMA,
            pltpu.SemaphoreType.DMA,
            pltpu.SemaphoreType.DMA,
        ],
        compiler_params=pltpu.CompilerParams(
            needs_layout_passes=False, use_tc_tiling_on_sc=False),
    )
    def body(env_hbm, i00_hbm, i01_hbm, i10_hbm, i11_hbm,
             e0_hbm, e1_hbm, wx_hbm, wy_hbm,
             le0_hbm, le1_hbm, le2_hbm,
             icA, evA, txA, wA, icB, evB, txB, wB, out_v,
             semA, semB, sinA, sinB, swA, swB):
        wid = lax.axis_index("s") * NC + lax.axis_index("c")
        base = wid * RW
        iota = lax.iota(jnp.int32, _LANES)
        corners = (i00_hbm, i01_hbm, i10_hbm, i11_hbm)
        evws = (e0_hbm, e1_hbm)
        sets = ((icA, evA, txA, wA, semA, sinA, swA),
                (icB, evB, txB, wB, semB, sinB, swB))

        def fire_ic(t, p):
            ic, ev, tx, w, sem, sin, sw = sets[p]
            b0 = base + t * C
            for corner in range(4):
                pltpu.async_copy(corners[corner].at[pl.ds(b0, C)],
                                 ic.at[corner], sin)

        def fire_w(t, p):
            ic, ev, tx, w, sem, sin, sw = sets[p]
            b0 = base + t * C
            pltpu.async_copy(e0_hbm.at[pl.ds(b0, C)], ev.at[0], sw)
            pltpu.async_copy(e1_hbm.at[pl.ds(b0, C)], ev.at[1], sw)
            pltpu.async_copy(wx_hbm.at[pl.ds(b0, C)], w.at[0], sw)
            pltpu.async_copy(wy_hbm.at[pl.ds(b0, C)], w.at[1], sw)

        def buildfire(p):
            ic, ev, tx, w, sem, sin, sw = sets[p]
            for corner in range(4):
                pltpu.make_async_copy(i00_hbm.at[pl.ds(0, C)],
                                      ic.at[corner], sin).wait()
            for corner in range(4):
                pltpu.async_copy(env_hbm.at[ic.at[corner]],
                                 tx.at[pl.ds(corner * C, C)], sem)

        def drain(p):
            ic, ev, tx, w, sem, sin, sw = sets[p]
            for j in range(4):
                pltpu.make_async_copy(env_hbm.at[pl.ds(0, C)],
                                      tx.at[pl.ds(j * C, C)], sem).wait()

        def combine_out(t, p):
            ic, ev, tx, w, sem, sin, sw = sets[p]
            b0 = base + t * C
            for j in range(2):
                pltpu.make_async_copy(e0_hbm.at[pl.ds(0, C)],
                                      ev.at[j], sw).wait()
            for j in range(2):
                pltpu.make_async_copy(wx_hbm.at[pl.ds(0, C)],
                                      w.at[j], sw).wait()

            def group(g, gcarry):
                s = pl.ds(g * _LANES, _LANES)
                ray = g * _LANES + iota
                e0v = ev[0, s]
                e1v = ev[1, s]
                wxv = w[0, s]
                wyv = w[1, s]
                c0 = e0v << 2
                isb = e1v >= 4
                off1 = jnp.where(isb, C, 0)
                c1 = jnp.where(isb, 0, e1v << 2)
                ra0 = ray
                rb0 = ray + off1
                ra1 = ray + 2 * C
                rb1 = ra1 + off1
                for ch in range(3):
                    v00 = plsc.load_gather(tx, [ra0, c0 + ch])
                    v01 = plsc.load_gather(tx, [rb0, c1 + ch])
                    v10 = plsc.load_gather(tx, [ra1, c0 + ch])
                    v11 = plsc.load_gather(tx, [rb1, c1 + ch])
                    top = v00 + wxv * (v01 - v00)
                    bot = v10 + wxv * (v11 - v10)
                    out_v[ch, s] = (top + wyv * (bot - top)) * (1.0 / 256.0)
                return gcarry

            lax.fori_loop(0, GROUPS, group, 0)
            pltpu.sync_copy(out_v.at[0], le0_hbm.at[pl.ds(b0, C)])
            pltpu.sync_copy(out_v.at[1], le1_hbm.at[pl.ds(b0, C)])
            pltpu.sync_copy(out_v.at[2], le2_hbm.at[pl.ds(b0, C)])

        fire_ic(0, 0)
        fire_w(0, 0)
        fire_ic(1, 1)
        fire_w(1, 1)
        buildfire(0)
        buildfire(1)

        def piter(i, carry):
            T = 2 * i
            drain(0)

            @pl.when(T + 2 < NCHUNK)
            def _():
                fire_ic(T + 2, 0)

            combine_out(T, 0)

            @pl.when(T + 2 < NCHUNK)
            def _():
                fire_w(T + 2, 0)
                buildfire(0)

            drain(1)

            @pl.when(T + 3 < NCHUNK)
            def _():
                fire_ic(T + 3, 1)

            combine_out(T + 1, 1)

            @pl.when(T + 3 < NCHUNK)
            def _():
                fire_w(T + 3, 1)
                buildfire(1)
            return carry

        lax.fori_loop(0, NCHUNK // 2, piter, 0)

    le0, le1, le2 = body(env16, i00, i01, i10, i11, e0, e1, wx, wy)
    return jnp.stack([le0, le1, le2], axis=-1)


def kernel(position, light_dir, envmap):
    B = light_dir.shape[0]
    H, W = envmap.shape[1], envmap.shape[2]
    ldT = light_dir.T
    i00, i01, i10, i11, e0, e1, wx, wy = _uv_kernel(ldT, H, W)
    env16 = jnp.concatenate(
        [jnp.transpose(envmap, (1, 2, 0)),
         jnp.zeros((H, W, 1), jnp.float32)], axis=-1).reshape(H * W // 4, 16)
    le = _sc_gather_combine(env16, i00, i01, i10, i11, e0, e1, wx, wy)
    pdf = jnp.full((B, 1), 1.0 / (4 * math.pi), dtype=jnp.float32)
    valid = jnp.ones((B, 1), dtype=bool)
    return (le, pdf, valid)


# speedup vs baseline: 48.9436x; 48.9436x over previous
import functools
import math

import jax
import jax.numpy as jnp
from jax import lax
from jax.experimental import pallas as pl
from jax.experimental.pallas import tpu as pltpu
from jax.experimental.pallas import tpu_sc as plsc


def _uv_body(W, H, ld_ref, i00_ref, i01_ref, i10_ref, i11_ref,
             e0_ref, e1_ref, wx_ref, wy_ref):
    x = ld_ref[0, :]
    y = ld_ref[1, :]
    z = ld_ref[2, :]
    norm = jnp.sqrt(x * x + y * y + z * z)
    yn = y / norm
    yc = jnp.clip(yn, -1.0 + 1e-06, 1.0 - 1e-06)
    theta = jnp.arctan2(jnp.sqrt((1.0 + yc) * (1.0 - yc)), yc)
    phi = jnp.arctan2(x, z)
    u = phi / (2.0 * math.pi) + 0.5
    u = u - jnp.floor(u)
    v = theta / math.pi
    xf = jnp.clip(u * W, 0.0, W - 1.0)
    yf = jnp.clip(v * H, 0.0, H - 1.0)
    x0f = jnp.floor(xf)
    y0f = jnp.floor(yf)
    wx_ref[...] = xf - x0f
    wy_ref[...] = yf - y0f
    x0 = x0f.astype(jnp.int32)
    y0 = y0f.astype(jnp.int32)
    x1 = jnp.minimum(x0 + 1, int(W) - 1)
    y1 = jnp.minimum(y0 + 1, int(H) - 1)
    W4 = int(W) // 4
    k = x0 >> 2
    kb = jnp.minimum(k + 1, W4 - 1)
    i00_ref[...] = y0 * W4 + k
    i01_ref[...] = y0 * W4 + kb
    i10_ref[...] = y1 * W4 + k
    i11_ref[...] = y1 * W4 + kb
    e0_ref[...] = x0 & 3
    e1_ref[...] = x1 - (k << 2)


def _uv_kernel(ldT, H, W, TB=8192):
    B = ldT.shape[1]
    G = B // TB
    iout = jax.ShapeDtypeStruct((B,), jnp.int32)
    fout = jax.ShapeDtypeStruct((B,), jnp.float32)
    ospec = pl.BlockSpec((TB,), lambda i: (i,))
    outs = pl.pallas_call(
        functools.partial(_uv_body, float(W), float(H)),
        grid=(G,),
        in_specs=[pl.BlockSpec((3, TB), lambda i: (0, i))],
        out_specs=[ospec] * 8,
        out_shape=[iout, iout, iout, iout, iout, iout, fout, fout],
    )(ldT)
    return outs


def _env16_kernel(envmap, M=4096):
    H, W = envmap.shape[1], envmap.shape[2]
    HW = H * W
    env_flat = envmap.reshape(3 * HW)
    info = plsc.get_sparse_core_info()
    NC, NS = info.num_cores, info.num_subcores
    NW = NC * NS
    TPW = HW // NW
    NCHUNK = TPW // M
    GROUPS = M // _LANES

    mesh = plsc.VectorSubcoreMesh(core_axis_name="c", subcore_axis_name="s")

    @functools.partial(
        pl.kernel,
        out_type=jax.ShapeDtypeStruct((HW // 4, 16), jnp.float32),
        mesh=mesh,
        scratch_types=[
            pltpu.VMEM((3, M), jnp.float32),
            pltpu.VMEM((M // 4, 16), jnp.float32),
            pltpu.SemaphoreType.DMA,
        ],
        compiler_params=pltpu.CompilerParams(
            needs_layout_passes=False, use_tc_tiling_on_sc=False),
    )
    def body(env_hbm, out_hbm, pv, ov, sem):
        wid = lax.axis_index("s") * NC + lax.axis_index("c")
        base = wid * TPW
        iota = lax.iota(jnp.int32, _LANES)

        def chunk(t, carry):
            t0 = base + t * M
            cps = [
                pltpu.async_copy(env_hbm.at[pl.ds(c * HW + t0, M)],
                                 pv.at[c], sem)
                for c in range(3)
            ]
            for cp in cps:
                cp.wait()
            zero = jnp.zeros((_LANES,), jnp.float32)

            def group(g, gcarry):
                s = pl.ds(g * _LANES, _LANES)
                t = g * _LANES + iota
                rows = t >> 2
                colbase = (t & 3) << 2
                for c in range(3):
                    plsc.store_scatter(ov, [rows, colbase + c], pv[c, s])
                plsc.store_scatter(ov, [rows, colbase + 3], zero)
                return gcarry

            lax.fori_loop(0, GROUPS, group, 0)
            pltpu.sync_copy(ov, out_hbm.at[pl.ds(t0 // 4, M // 4)])
            return carry

        lax.fori_loop(0, NCHUNK, chunk, 0)

    return body(env_flat)


_LANES = 16


def _sc_gather_combine(env16, i00, i01, i10, i11, e0, e1, wx, wy, C=512):
    B = i00.shape[0]
    info = plsc.get_sparse_core_info()
    NC, NS = info.num_cores, info.num_subcores
    NW = NC * NS
    RW = B // NW
    NCHUNK = RW // C
    GROUPS = C // _LANES
    assert NCHUNK % 2 == 0

    mesh = plsc.VectorSubcoreMesh(core_axis_name="c", subcore_axis_name="s")
    fout = jax.ShapeDtypeStruct((B,), jnp.float32)

    buf_set = [
        pltpu.VMEM((4, C), jnp.int32),
        pltpu.VMEM((2, C), jnp.int32),
        pltpu.VMEM((4 * C, 16), jnp.float32),
        pltpu.VMEM((2, C), jnp.float32),
    ]

    @functools.partial(
        pl.kernel,
        out_type=[fout, fout, fout],
        mesh=mesh,
        scratch_types=buf_set + buf_set + [
            pltpu.VMEM((3, C), jnp.float32),
            pltpu.SemaphoreType.DMA,
            pltpu.SemaphoreType.DMA,
            pltpu.SemaphoreType.DMA,
            pltpu.SemaphoreType.DMA,
            pltpu.SemaphoreType.DMA,
            pltpu.SemaphoreType.DMA,
        ],
        compiler_params=pltpu.CompilerParams(
            needs_layout_passes=False, use_tc_tiling_on_sc=False),
    )
    def body(env_hbm, i00_hbm, i01_hbm, i10_hbm, i11_hbm,
             e0_hbm, e1_hbm, wx_hbm, wy_hbm,
             le0_hbm, le1_hbm, le2_hbm,
             icA, evA, txA, wA, icB, evB, txB, wB, out_v,
             semA, semB, sinA, sinB, swA, swB):
        wid = lax.axis_index("s") * NC + lax.axis_index("c")
        base = wid * RW
        iota = lax.iota(jnp.int32, _LANES)
        corners = (i00_hbm, i01_hbm, i10_hbm, i11_hbm)
        evws = (e0_hbm, e1_hbm)
        sets = ((icA, evA, txA, wA, semA, sinA, swA),
                (icB, evB, txB, wB, semB, sinB, swB))

        def fire_ic(t, p):
            ic, ev, tx, w, sem, sin, sw = sets[p]
            b0 = base + t * C
            for corner in range(4):
                pltpu.async_copy(corners[corner].at[pl.ds(b0, C)],
                                 ic.at[corner], sin)

        def fire_w(t, p):
            ic, ev, tx, w, sem, sin, sw = sets[p]
            b0 = base + t * C
            pltpu.async_copy(e0_hbm.at[pl.ds(b0, C)], ev.at[0], sw)
            pltpu.async_copy(e1_hbm.at[pl.ds(b0, C)], ev.at[1], sw)
            pltpu.async_copy(wx_hbm.at[pl.ds(b0, C)], w.at[0], sw)
            pltpu.async_copy(wy_hbm.at[pl.ds(b0, C)], w.at[1], sw)

        def buildfire(p):
            ic, ev, tx, w, sem, sin, sw = sets[p]
            for corner in range(4):
                pltpu.make_async_copy(i00_hbm.at[pl.ds(0, C)],
                                      ic.at[corner], sin).wait()
            for corner in range(4):
                pltpu.async_copy(env_hbm.at[ic.at[corner]],
                                 tx.at[pl.ds(corner * C, C)], sem)

        def drain(p):
            ic, ev, tx, w, sem, sin, sw = sets[p]
            for j in range(4):
                pltpu.make_async_copy(env_hbm.at[pl.ds(0, C)],
                                      tx.at[pl.ds(j * C, C)], sem).wait()

        def combine_out(t, p):
            ic, ev, tx, w, sem, sin, sw = sets[p]
            b0 = base + t * C
            for j in range(2):
                pltpu.make_async_copy(e0_hbm.at[pl.ds(0, C)],
                                      ev.at[j], sw).wait()
            for j in range(2):
                pltpu.make_async_copy(wx_hbm.at[pl.ds(0, C)],
                                      w.at[j], sw).wait()

            def group(g, gcarry):
                s = pl.ds(g * _LANES, _LANES)
                ray = g * _LANES + iota
                e0v = ev[0, s]
                e1v = ev[1, s]
                wxv = w[0, s]
                wyv = w[1, s]
                c0 = e0v << 2
                isb = e1v >= 4
                off1 = jnp.where(isb, C, 0)
                c1 = jnp.where(isb, 0, e1v << 2)
                ra0 = ray
                rb0 = ray + off1
                ra1 = ray + 2 * C
                rb1 = ra1 + off1
                for ch in range(3):
                    v00 = plsc.load_gather(tx, [ra0, c0 + ch])
                    v01 = plsc.load_gather(tx, [rb0, c1 + ch])
                    v10 = plsc.load_gather(tx, [ra1, c0 + ch])
                    v11 = plsc.load_gather(tx, [rb1, c1 + ch])
                    top = v00 + wxv * (v01 - v00)
                    bot = v10 + wxv * (v11 - v10)
                    out_v[ch, s] = (top + wyv * (bot - top)) * (1.0 / 256.0)
                return gcarry

            lax.fori_loop(0, GROUPS, group, 0)
            pltpu.sync_copy(out_v.at[0], le0_hbm.at[pl.ds(b0, C)])
            pltpu.sync_copy(out_v.at[1], le1_hbm.at[pl.ds(b0, C)])
            pltpu.sync_copy(out_v.at[2], le2_hbm.at[pl.ds(b0, C)])

        fire_ic(0, 0)
        fire_w(0, 0)
        fire_ic(1, 1)
        fire_w(1, 1)
        buildfire(0)
        buildfire(1)

        def piter(i, carry):
            T = 2 * i
            drain(0)

            @pl.when(T + 2 < NCHUNK)
            def _():
                fire_ic(T + 2, 0)

            combine_out(T, 0)

            @pl.when(T + 2 < NCHUNK)
            def _():
                fire_w(T + 2, 0)
                buildfire(0)

            drain(1)

            @pl.when(T + 3 < NCHUNK)
            def _():
                fire_ic(T + 3, 1)

            combine_out(T + 1, 1)

            @pl.when(T + 3 < NCHUNK)
            def _():
                fire_w(T + 3, 1)
                buildfire(1)
            return carry

        lax.fori_loop(0, NCHUNK // 2, piter, 0)

    le0, le1, le2 = body(env16, i00, i01, i10, i11, e0, e1, wx, wy)
    return jnp.stack([le0, le1, le2], axis=-1)


def kernel(position, light_dir, envmap):
    B = light_dir.shape[0]
    H, W = envmap.shape[1], envmap.shape[2]
    ldT = light_dir.T
    i00, i01, i10, i11, e0, e1, wx, wy = _uv_kernel(ldT, H, W)
    env16 = _env16_kernel(envmap)
    le = _sc_gather_combine(env16, i00, i01, i10, i11, e0, e1, wx, wy)
    pdf = jnp.full((B, 1), 1.0 / (4 * math.pi), dtype=jnp.float32)
    valid = jnp.ones((B, 1), dtype=bool)
    return (le, pdf, valid)


# speedup vs baseline: 61.9679x; 1.2661x over previous
import functools
import math

import jax
import jax.numpy as jnp
from jax import lax
from jax.experimental import pallas as pl
from jax.experimental.pallas import tpu as pltpu
from jax.experimental.pallas import tpu_sc as plsc


def _uv_body(W, H, ld_ref, i00_ref, i01_ref, i10_ref, i11_ref,
             e0_ref, e1_ref, wx_ref, wy_ref):
    x = ld_ref[0, :]
    y = ld_ref[1, :]
    z = ld_ref[2, :]
    norm = jnp.sqrt(x * x + y * y + z * z)
    yn = y / norm
    yc = jnp.clip(yn, -1.0 + 1e-06, 1.0 - 1e-06)
    theta = jnp.arctan2(jnp.sqrt((1.0 + yc) * (1.0 - yc)), yc)
    phi = jnp.arctan2(x, z)
    u = phi / (2.0 * math.pi) + 0.5
    u = u - jnp.floor(u)
    v = theta / math.pi
    xf = jnp.clip(u * W, 0.0, W - 1.0)
    yf = jnp.clip(v * H, 0.0, H - 1.0)
    x0f = jnp.floor(xf)
    y0f = jnp.floor(yf)
    wx_ref[...] = xf - x0f
    wy_ref[...] = yf - y0f
    x0 = x0f.astype(jnp.int32)
    y0 = y0f.astype(jnp.int32)
    x1 = jnp.minimum(x0 + 1, int(W) - 1)
    y1 = jnp.minimum(y0 + 1, int(H) - 1)
    W4 = int(W) // 4
    k = x0 >> 2
    kb = jnp.minimum(k + 1, W4 - 1)
    i00_ref[...] = y0 * W4 + k
    i01_ref[...] = y0 * W4 + kb
    i10_ref[...] = y1 * W4 + k
    i11_ref[...] = y1 * W4 + kb
    e0_ref[...] = x0 & 3
    e1_ref[...] = x1 - (k << 2)


def _uv_kernel(ldT, H, W, TB=8192):
    B = ldT.shape[1]
    G = B // TB
    iout = jax.ShapeDtypeStruct((B,), jnp.int32)
    fout = jax.ShapeDtypeStruct((B,), jnp.float32)
    ospec = pl.BlockSpec((TB,), lambda i: (i,))
    outs = pl.pallas_call(
        functools.partial(_uv_body, float(W), float(H)),
        grid=(G,),
        in_specs=[pl.BlockSpec((3, TB), lambda i: (0, i))],
        out_specs=[ospec] * 8,
        out_shape=[iout, iout, iout, iout, iout, iout, fout, fout],
    )(ldT)
    return outs


def _env16_kernel(envmap, M=4096):
    H, W = envmap.shape[1], envmap.shape[2]
    HW = H * W
    env_flat = envmap.reshape(3 * HW)
    info = plsc.get_sparse_core_info()
    NC, NS = info.num_cores, info.num_subcores
    NW = NC * NS
    TPW = HW // NW
    NCHUNK = TPW // M
    GROUPS = M // _LANES

    mesh = plsc.VectorSubcoreMesh(core_axis_name="c", subcore_axis_name="s")

    assert NCHUNK % 2 == 0

    @functools.partial(
        pl.kernel,
        out_type=jax.ShapeDtypeStruct((HW // 4, 16), jnp.float32),
        mesh=mesh,
        scratch_types=[
            pltpu.VMEM((3, M), jnp.float32),
            pltpu.VMEM((M // 4, 16), jnp.float32),
            pltpu.VMEM((3, M), jnp.float32),
            pltpu.VMEM((M // 4, 16), jnp.float32),
            pltpu.SemaphoreType.DMA,
            pltpu.SemaphoreType.DMA,
            pltpu.SemaphoreType.DMA,
            pltpu.SemaphoreType.DMA,
        ],
        compiler_params=pltpu.CompilerParams(
            needs_layout_passes=False, use_tc_tiling_on_sc=False),
    )
    def body(env_hbm, out_hbm, pvA, ovA, pvB, ovB, sinA, soutA, sinB, soutB):
        wid = lax.axis_index("s") * NC + lax.axis_index("c")
        base = wid * TPW
        iota = lax.iota(jnp.int32, _LANES)
        sets = ((pvA, ovA, sinA, soutA), (pvB, ovB, sinB, soutB))

        def fire_in(t, p):
            pv, ov, sin, sout = sets[p]
            t0 = base + t * M
            for c in range(3):
                pltpu.async_copy(env_hbm.at[pl.ds(c * HW + t0, M)],
                                 pv.at[c], sin)

        def process(t, p):
            pv, ov, sin, sout = sets[p]
            t0 = base + t * M
            for c in range(3):
                pltpu.make_async_copy(env_hbm.at[pl.ds(0, M)],
                                      pv.at[c], sin).wait()

            @pl.when(t >= 2)
            def _():
                pltpu.make_async_copy(ov, out_hbm.at[pl.ds(0, M // 4)],
                                      sout).wait()

            def group(g, gcarry):
                s = pl.ds(g * _LANES, _LANES)
                tt = g * _LANES + iota
                rows = tt >> 2
                colbase = (tt & 3) << 2
                for c in range(3):
                    plsc.store_scatter(ov, [rows, colbase + c], pv[c, s])
                return gcarry

            lax.fori_loop(0, GROUPS, group, 0)
            pltpu.async_copy(ov, out_hbm.at[pl.ds(t0 // 4, M // 4)], sout)

            @pl.when(t + 2 < NCHUNK)
            def _():
                fire_in(t + 2, p)

        fire_in(0, 0)
        fire_in(1, 1)

        def piter(i, carry):
            T = 2 * i
            process(T, 0)
            process(T + 1, 1)
            return carry

        lax.fori_loop(0, NCHUNK // 2, piter, 0)
        for p in range(2):
            pv, ov, sin, sout = sets[p]
            pltpu.make_async_copy(ov, out_hbm.at[pl.ds(0, M // 4)],
                                  sout).wait()

    return body(env_flat)


_LANES = 16


def _sc_gather_combine(env16, i00, i01, i10, i11, e0, e1, wx, wy, C=512):
    B = i00.shape[0]
    info = plsc.get_sparse_core_info()
    NC, NS = info.num_cores, info.num_subcores
    NW = NC * NS
    RW = B // NW
    NCHUNK = RW // C
    GROUPS = C // _LANES
    assert NCHUNK % 2 == 0

    mesh = plsc.VectorSubcoreMesh(core_axis_name="c", subcore_axis_name="s")
    fout = jax.ShapeDtypeStruct((B,), jnp.float32)

    buf_set = [
        pltpu.VMEM((4, C), jnp.int32),
        pltpu.VMEM((2, C), jnp.int32),
        pltpu.VMEM((4 * C, 16), jnp.float32),
        pltpu.VMEM((2, C), jnp.float32),
    ]

    @functools.partial(
        pl.kernel,
        out_type=[fout, fout, fout],
        mesh=mesh,
        scratch_types=buf_set + buf_set + [
            pltpu.VMEM((3, C), jnp.float32),
            pltpu.SemaphoreType.DMA,
            pltpu.SemaphoreType.DMA,
            pltpu.SemaphoreType.DMA,
            pltpu.SemaphoreType.DMA,
            pltpu.SemaphoreType.DMA,
            pltpu.SemaphoreType.DMA,
        ],
        compiler_params=pltpu.CompilerParams(
            needs_layout_passes=False, use_tc_tiling_on_sc=False),
    )
    def body(env_hbm, i00_hbm, i01_hbm, i10_hbm, i11_hbm,
             e0_hbm, e1_hbm, wx_hbm, wy_hbm,
             le0_hbm, le1_hbm, le2_hbm,
             icA, evA, txA, wA, icB, evB, txB, wB, out_v,
             semA, semB, sinA, sinB, swA, swB):
        wid = lax.axis_index("s") * NC + lax.axis_index("c")
        base = wid * RW
        iota = lax.iota(jnp.int32, _LANES)
        corners = (i00_hbm, i01_hbm, i10_hbm, i11_hbm)
        evws = (e0_hbm, e1_hbm)
        sets = ((icA, evA, txA, wA, semA, sinA, swA),
                (icB, evB, txB, wB, semB, sinB, swB))

        def fire_ic(t, p):
            ic, ev, tx, w, sem, sin, sw = sets[p]
            b0 = base + t * C
            for corner in range(4):
                pltpu.async_copy(corners[corner].at[pl.ds(b0, C)],
                                 ic.at[corner], sin)

        def fire_w(t, p):
            ic, ev, tx, w, sem, sin, sw = sets[p]
            b0 = base + t * C
            pltpu.async_copy(e0_hbm.at[pl.ds(b0, C)], ev.at[0], sw)
            pltpu.async_copy(e1_hbm.at[pl.ds(b0, C)], ev.at[1], sw)
            pltpu.async_copy(wx_hbm.at[pl.ds(b0, C)], w.at[0], sw)
            pltpu.async_copy(wy_hbm.at[pl.ds(b0, C)], w.at[1], sw)

        def buildfire(p):
            ic, ev, tx, w, sem, sin, sw = sets[p]
            for corner in range(4):
                pltpu.make_async_copy(i00_hbm.at[pl.ds(0, C)],
                                      ic.at[corner], sin).wait()
            for corner in range(4):
                pltpu.async_copy(env_hbm.at[ic.at[corner]],
                                 tx.at[pl.ds(corner * C, C)], sem)

        def drain(p):
            ic, ev, tx, w, sem, sin, sw = sets[p]
            for j in range(4):
                pltpu.make_async_copy(env_hbm.at[pl.ds(0, C)],
                                      tx.at[pl.ds(j * C, C)], sem).wait()

        def combine_out(t, p):
            ic, ev, tx, w, sem, sin, sw = sets[p]
            b0 = base + t * C
            for j in range(2):
                pltpu.make_async_copy(e0_hbm.at[pl.ds(0, C)],
                                      ev.at[j], sw).wait()
            for j in range(2):
                pltpu.make_async_copy(wx_hbm.at[pl.ds(0, C)],
                                      w.at[j], sw).wait()

            def group(g, gcarry):
                s = pl.ds(g * _LANES, _LANES)
                ray = g * _LANES + iota
                e0v = ev[0, s]
                e1v = ev[1, s]
                wxv = w[0, s]
                wyv = w[1, s]
                c0 = e0v << 2
                isb = e1v >= 4
                off1 = jnp.where(isb, C, 0)
                c1 = jnp.where(isb, 0, e1v << 2)
                ra0 = ray
                rb0 = ray + off1
                ra1 = ray + 2 * C
                rb1 = ra1 + off1
                for ch in range(3):
                    v00 = plsc.load_gather(tx, [ra0, c0 + ch])
                    v01 = plsc.load_gather(tx, [rb0, c1 + ch])
                    v10 = plsc.load_gather(tx, [ra1, c0 + ch])
                    v11 = plsc.load_gather(tx, [rb1, c1 + ch])
                    top = v00 + wxv * (v01 - v00)
                    bot = v10 + wxv * (v11 - v10)
                    out_v[ch, s] = (top + wyv * (bot - top)) * (1.0 / 256.0)
                return gcarry

            lax.fori_loop(0, GROUPS, group, 0)
            pltpu.sync_copy(out_v.at[0], le0_hbm.at[pl.ds(b0, C)])
            pltpu.sync_copy(out_v.at[1], le1_hbm.at[pl.ds(b0, C)])
            pltpu.sync_copy(out_v.at[2], le2_hbm.at[pl.ds(b0, C)])

        fire_ic(0, 0)
        fire_w(0, 0)
        fire_ic(1, 1)
        fire_w(1, 1)
        buildfire(0)
        buildfire(1)

        def piter(i, carry):
            T = 2 * i
            drain(0)

            @pl.when(T + 2 < NCHUNK)
            def _():
                fire_ic(T + 2, 0)

            combine_out(T, 0)

            @pl.when(T + 2 < NCHUNK)
            def _():
                fire_w(T + 2, 0)
                buildfire(0)

            drain(1)

            @pl.when(T + 3 < NCHUNK)
            def _():
                fire_ic(T + 3, 1)

            combine_out(T + 1, 1)

            @pl.when(T + 3 < NCHUNK)
            def _():
                fire_w(T + 3, 1)
                buildfire(1)
            return carry

        lax.fori_loop(0, NCHUNK // 2, piter, 0)

    le0, le1, le2 = body(env16, i00, i01, i10, i11, e0, e1, wx, wy)
    return jnp.stack([le0, le1, le2], axis=-1)


def kernel(position, light_dir, envmap):
    B = light_dir.shape[0]
    H, W = envmap.shape[1], envmap.shape[2]
    ldT = light_dir.T
    i00, i01, i10, i11, e0, e1, wx, wy = _uv_kernel(ldT, H, W)
    env16 = _env16_kernel(envmap)
    le = _sc_gather_combine(env16, i00, i01, i10, i11, e0, e1, wx, wy)
    pdf = jnp.full((B, 1), 1.0 / (4 * math.pi), dtype=jnp.float32)
    valid = jnp.ones((B, 1), dtype=bool)
    return (le, pdf, valid)
